# baseline (speedup 1.0000x reference)
import jax
import jax.numpy as jnp
from jax.experimental import pallas as pl
from jax.experimental.pallas import tpu as pltpu

N = 10000
NFEAT = 128
NHID = 16
NCLASS = 8

TI = 400
NI = N // TI
NC = 2
NSTEPS = 1 + 2 * NI


def _adj_index(g):
    j = g - (NI + 1)
    c_idx = jnp.where(j == 0, NI - 1,
                      jnp.where(j <= 2 * NC, NC + (j - 1) // 2, j - 1))
    return (jnp.where(g == 0, 0,
                      jnp.where(g <= NI, g - 1, c_idx)), 0)


def _out_index(g):
    j = g - (NI + 1)
    is_cache = (j >= 2) & (j <= 2 * NC) & (j % 2 == 0)
    c_idx = jnp.where(j <= 0, NI - 1,
                      jnp.where(is_cache, j // 2 - 1,
                                jnp.where(j <= 2 * NC, NC + (j - 1) // 2,
                                          j - 1)))
    return (c_idx, 0)


def _log_softmax(z):
    m = jnp.max(z, axis=1, keepdims=True)
    return z - (jnp.log(jnp.sum(jnp.exp(z - m), axis=1, keepdims=True)) + m)


def _body(adj_ref, x_ref, w1_ref, w2_ref, b1_ref, b2_ref, o_ref,
          s1_ref, s2_ref, cache_ref):
    g = pl.program_id(0)

    @pl.when(g == 0)
    def _():
        s1_ref[...] = jnp.dot(x_ref[...], w1_ref[...],
                              preferred_element_type=jnp.float32)

    @pl.when((g >= 1) & (g <= NI))
    def _():
        h = jnp.maximum(jnp.dot(adj_ref[...], s1_ref[...],
                                preferred_element_type=jnp.float32)
                        + b1_ref[...], 0.0)
        s2_ref[pl.ds((g - 1) * TI, TI), :] = jnp.dot(
            h, w2_ref[...], preferred_element_type=jnp.float32)

        @pl.when(g <= NC)
        def _():
            cache_ref[pl.ds((g - 1) * TI, TI), :] = adj_ref[...].astype(
                jnp.bfloat16)

    @pl.when(g > NI)
    def _():
        j = g - (NI + 1)
        use_cache = (j >= 2) & (j <= 2 * NC) & (j % 2 == 0)

        @pl.when(use_cache)
        def _():
            blk = cache_ref[pl.ds((j // 2 - 1) * TI, TI), :]
            o_ref[...] = _log_softmax(
                jnp.dot(blk, s2_ref[...].astype(jnp.bfloat16),
                        preferred_element_type=jnp.float32) + b2_ref[...])

        @pl.when(jnp.logical_not(use_cache))
        def _():
            o_ref[...] = _log_softmax(
                jnp.dot(adj_ref[...], s2_ref[...],
                        preferred_element_type=jnp.float32) + b2_ref[...])


def kernel(x, adj, W1, b1, W2, b2):
    return pl.pallas_call(
        _body,
        grid=(NSTEPS,),
        in_specs=[
            pl.BlockSpec((TI, N), _adj_index),
            pl.BlockSpec((N, NFEAT), lambda g: (0, 0)),
            pl.BlockSpec((NFEAT, NHID), lambda g: (0, 0)),
            pl.BlockSpec((NHID, NCLASS), lambda g: (0, 0)),
            pl.BlockSpec((1, NHID), lambda g: (0, 0)),
            pl.BlockSpec((1, NCLASS), lambda g: (0, 0)),
        ],
        out_specs=pl.BlockSpec((TI, NCLASS), _out_index),
        out_shape=jax.ShapeDtypeStruct((N, NCLASS), jnp.float32),
        scratch_shapes=[
            pltpu.VMEM((N, NHID), jnp.float32),
            pltpu.VMEM((N, NCLASS), jnp.float32),
            pltpu.VMEM((NC * TI, N), jnp.bfloat16),
        ],
        compiler_params=pltpu.CompilerParams(
            dimension_semantics=("arbitrary",),
            vmem_limit_bytes=100 * 1024 * 1024),
    )(adj, x, W1, W2, b1.reshape(1, NHID), b2.reshape(1, NCLASS))


# speedup vs baseline: 1.0169x; 1.0169x over previous
import jax
import jax.numpy as jnp
from jax.experimental import pallas as pl
from jax.experimental.pallas import tpu as pltpu

N = 10000
NFEAT = 128
NHID = 16
NCLASS = 8

TI = 400
NI = N // TI
NC = 3
TX = 2000
NX = N // TX
NSTEPS = NX + 2 * NI


def _adj_index(g):
    j = g - (NX + NI)
    c_idx = jnp.where(j <= NC, NI - 1, j - 1)
    return (jnp.where(g < NX, 0,
                      jnp.where(j < 0, g - NX, c_idx)), 0)


def _x_index(g):
    return (jnp.where(g < NX, g, NX - 1), 0)


def _out_index(g):
    j = g - (NX + NI)
    return (jnp.where(j <= 0, NI - 1, j - 1), 0)


def _log_softmax(z):
    m = jnp.max(z, axis=1, keepdims=True)
    return z - (jnp.log(jnp.sum(jnp.exp(z - m), axis=1, keepdims=True)) + m)


def _body(adj_ref, x_ref, w1_ref, w2_ref, b1_ref, b2_ref, o_ref,
          s1_ref, s2_ref, cache_ref):
    g = pl.program_id(0)

    @pl.when(g < NX)
    def _():
        s1_ref[pl.ds(g * TX, TX), :] = jnp.dot(
            x_ref[...], w1_ref[...],
            preferred_element_type=jnp.float32).astype(jnp.bfloat16)

    @pl.when((g >= NX) & (g < NX + NI))
    def _():
        b = g - NX
        h = jnp.maximum(jnp.dot(adj_ref[...], s1_ref[...],
                                preferred_element_type=jnp.float32)
                        + b1_ref[...], 0.0)
        s2_ref[pl.ds(b * TI, TI), :] = jnp.dot(
            h, w2_ref[...],
            preferred_element_type=jnp.float32).astype(jnp.bfloat16)

        @pl.when(b < NC)
        def _():
            cache_ref[pl.ds(b * TI, TI), :] = adj_ref[...].astype(
                jnp.bfloat16)

    @pl.when(g >= NX + NI)
    def _():
        j = g - (NX + NI)
        use_cache = (j >= 1) & (j <= NC)

        @pl.when(use_cache)
        def _():
            blk = cache_ref[pl.ds((j - 1) * TI, TI), :]
            o_ref[...] = _log_softmax(
                jnp.dot(blk, s2_ref[...],
                        preferred_element_type=jnp.float32) + b2_ref[...])

        @pl.when(jnp.logical_not(use_cache))
        def _():
            o_ref[...] = _log_softmax(
                jnp.dot(adj_ref[...], s2_ref[...],
                        preferred_element_type=jnp.float32) + b2_ref[...])


def kernel(x, adj, W1, b1, W2, b2):
    return pl.pallas_call(
        _body,
        grid=(NSTEPS,),
        in_specs=[
            pl.BlockSpec((TI, N), _adj_index),
            pl.BlockSpec((TX, NFEAT), _x_index),
            pl.BlockSpec((NFEAT, NHID), lambda g: (0, 0)),
            pl.BlockSpec((NHID, NCLASS), lambda g: (0, 0)),
            pl.BlockSpec((1, NHID), lambda g: (0, 0)),
            pl.BlockSpec((1, NCLASS), lambda g: (0, 0)),
        ],
        out_specs=pl.BlockSpec((TI, NCLASS), _out_index),
        out_shape=jax.ShapeDtypeStruct((N, NCLASS), jnp.float32),
        scratch_shapes=[
            pltpu.VMEM((N, NHID), jnp.bfloat16),
            pltpu.VMEM((N, NCLASS), jnp.bfloat16),
            pltpu.VMEM((NC * TI, N), jnp.bfloat16),
        ],
        compiler_params=pltpu.CompilerParams(
            dimension_semantics=("arbitrary",),
            vmem_limit_bytes=100 * 1024 * 1024),
    )(adj, x, W1, W2, b1.reshape(1, NHID), b2.reshape(1, NCLASS))


# speedup vs baseline: 1.0389x; 1.0216x over previous
import jax
import jax.numpy as jnp
from jax.experimental import pallas as pl
from jax.experimental.pallas import tpu as pltpu

N = 10000
NFEAT = 128
NHID = 16
NCLASS = 8

TI = 200
NI = N // TI
D = 3
NCB = 3
TC = 208
NF = NI - D - NCB


def _log_softmax(z):
    m = jnp.max(z, axis=1, keepdims=True)
    return z - (jnp.log(jnp.sum(jnp.exp(z - m), axis=1, keepdims=True)) + m)


def _body(adj_hbm, x_ref, w1_ref, w2_ref, b1_ref, b2_ref, o_ref,
          s1_ref, s2_ref, cache_ref, buf0, buf1, buf2, sem0, sem1, sem2):
    bufs = (buf0, buf1, buf2)
    sems = (sem0, sem1, sem2)

    def copy(block_start, slot):
        return pltpu.make_async_copy(
            adj_hbm.at[pl.ds(block_start * TI, TI), :], bufs[slot],
            sems[slot])

    for d in range(D):
        copy(d, d).start()
    s1_ref[...] = jnp.dot(x_ref[...], w1_ref[...],
                          preferred_element_type=jnp.float32).astype(
                              jnp.bfloat16)

    def b_step(i, slot):
        copy(i, slot).wait()
        blk = bufs[slot][...]
        h = jnp.maximum(jnp.dot(blk, s1_ref[...],
                                preferred_element_type=jnp.float32)
                        + b1_ref[...], 0.0)
        s2_ref[pl.ds(i * TI, TI), :] = jnp.dot(
            h, w2_ref[...], preferred_element_type=jnp.float32)

        @pl.when(i < NCB)
        def _():
            cache_ref[pl.ds(i * TC, TI), :] = blk.astype(jnp.bfloat16)

        @pl.when(i + D < NI)
        def _():
            copy(i + D, slot).start()

    def b_loop(k, carry):
        for d in range(D):
            b_step(k * D + d, d)
        return carry

    nb_main = (NI // D) * D
    jax.lax.fori_loop(0, NI // D, b_loop, 0)
    for i in range(nb_main, NI):
        b_step(i, i % D)

    def emit(block_idx, src):
        z = jnp.dot(src, s2_ref[...],
                    preferred_element_type=jnp.float32) + b2_ref[...]
        o_ref[pl.ds(block_idx * TI, TI), :] = _log_softmax(z)

    for t in range(D):
        blk_id = NI - 1 - t
        slot = blk_id % D
        emit(blk_id, bufs[slot][...])
        if t < NF:
            copy(NCB + t, slot).start()

    def c_cache(m, carry):
        z = jnp.dot(cache_ref[pl.ds(m * TC, TI), :],
                    s2_ref[...].astype(jnp.bfloat16),
                    preferred_element_type=jnp.float32) + b2_ref[...]
        o_ref[pl.ds(m * TI, TI), :] = _log_softmax(z)
        return carry

    jax.lax.fori_loop(0, NCB, c_cache, 0)

    def f_step(b, slot):
        copy(b, slot).wait()
        emit(b, bufs[slot][...])

        @pl.when(b + D < NCB + NF)
        def _():
            copy(b + D, slot).start()

    def f_loop(k, carry):
        for d in range(D):
            b = NCB + k * D + d
            f_step(b, (NI - 1 - d) % D)
        return carry

    nf_main = (NF // D) * D
    jax.lax.fori_loop(0, NF // D, f_loop, 0)
    for j in range(nf_main, NF):
        f_step(NCB + j, (NI - 1 - j) % D)


def kernel(x, adj, W1, b1, W2, b2):
    return pl.pallas_call(
        _body,
        in_specs=[
            pl.BlockSpec(memory_space=pl.ANY),
            pl.BlockSpec(memory_space=pltpu.MemorySpace.VMEM),
            pl.BlockSpec(memory_space=pltpu.MemorySpace.VMEM),
            pl.BlockSpec(memory_space=pltpu.MemorySpace.VMEM),
            pl.BlockSpec(memory_space=pltpu.MemorySpace.VMEM),
            pl.BlockSpec(memory_space=pltpu.MemorySpace.VMEM),
        ],
        out_specs=pl.BlockSpec(memory_space=pltpu.MemorySpace.VMEM),
        out_shape=jax.ShapeDtypeStruct((N, NCLASS), jnp.float32),
        scratch_shapes=[
            pltpu.VMEM((N, NHID), jnp.bfloat16),
            pltpu.VMEM((N, NCLASS), jnp.float32),
            pltpu.VMEM((NCB * TC, N), jnp.bfloat16),
            pltpu.VMEM((TI, N), jnp.float32),
            pltpu.VMEM((TI, N), jnp.float32),
            pltpu.VMEM((TI, N), jnp.float32),
            pltpu.SemaphoreType.DMA,
            pltpu.SemaphoreType.DMA,
            pltpu.SemaphoreType.DMA,
        ],
        compiler_params=pltpu.CompilerParams(
            vmem_limit_bytes=100 * 1024 * 1024),
    )(adj, x, W1, W2, b1.reshape(1, NHID), b2.reshape(1, NCLASS))
